# baseline (speedup 1.0000x reference)
import functools

import jax
import jax.numpy as jnp
import numpy as np
from jax.experimental import pallas as pl
from jax.experimental.pallas import tpu as pltpu

ANCHOR_GAIN = 2.0
EPS = 1e-7


def _atan_pos(x):
    inv = x > 1.0
    r = jnp.where(inv, 1.0 / jnp.maximum(x, 1e-30), x)
    z = r * r
    p = 0.99997726 + z * (-0.33262347 + z * (0.19354346 + z * (
        -0.11643287 + z * (0.05265332 + z * -0.01172120))))
    a = r * p
    return jnp.where(inv, (np.pi / 2.0) - a, a)


def _loss_kernel(pred_hbm, tgt_hbm, mask_ref, anchors_ref, out_ref,
                 aobj, acls, anp, abox, pbuf, tbuf, psem, tsem,
                 *, n_steps, n_obj, n_cls, n_anchors):
    step = pl.program_id(0)
    b = step // n_anchors
    a = step % n_anchors

    slot = jax.lax.rem(step, 2)
    nslot = jax.lax.rem(step + 1, 2)

    @pl.when(step == 0)
    def _first():
        pltpu.make_async_copy(pred_hbm.at[0, 0], pbuf.at[0], psem.at[0]).start()
        pltpu.make_async_copy(tgt_hbm.at[0, 0], tbuf.at[0], tsem.at[0]).start()

    @pl.when(step + 1 < n_steps)
    def _prefetch():
        b1 = (step + 1) // n_anchors
        a1 = (step + 1) % n_anchors
        pltpu.make_async_copy(pred_hbm.at[b1, a1], pbuf.at[nslot],
                              psem.at[nslot]).start()
        pltpu.make_async_copy(tgt_hbm.at[b1, a1], tbuf.at[nslot],
                              tsem.at[nslot]).start()

    pltpu.make_async_copy(pred_hbm.at[b, a], pbuf.at[slot], psem.at[slot]).wait()
    pltpu.make_async_copy(tgt_hbm.at[b, a], tbuf.at[slot], tsem.at[slot]).wait()

    @pl.when(step == 0)
    def _init():
        aobj[...] = jnp.zeros_like(aobj)
        acls[...] = jnp.zeros_like(acls)
        anp[...] = jnp.zeros_like(anp)
        abox[...] = jnp.zeros_like(abox)

    z = pbuf[slot]
    t = tbuf[slot]
    m = mask_ref[0, 0].astype(jnp.float32)

    aw = anchors_ref[a, 2]
    ah = anchors_ref[a, 3]

    zc = z[4:]
    bce = (jnp.maximum(zc, 0.0) + jnp.log1p(jnp.exp(-jnp.abs(zc)))) - zc * t[4:]
    aobj[...] += bce[0]
    acls[...] += jnp.sum(bce[1:], axis=0) * m
    anp[...] += m

    G = ANCHOR_GAIN
    px = jax.nn.sigmoid(z[0]) * G - (G - 1.0) / 2.0
    py = jax.nn.sigmoid(z[1]) * G - (G - 1.0) / 2.0
    pw = (jax.nn.sigmoid(z[2]) * G) ** 2 * aw
    ph = (jax.nn.sigmoid(z[3]) * G) ** 2 * ah
    tx = t[0]
    ty = t[1]
    tw = t[2]
    th = t[3]

    b1x1 = px - pw * 0.5
    b1x2 = px + pw * 0.5
    b1y1 = py - ph * 0.5
    b1y2 = py + ph * 0.5
    b2x1 = tx - tw * 0.5
    b2x2 = tx + tw * 0.5
    b2y1 = ty - th * 0.5
    b2y2 = ty + th * 0.5
    inter = (jnp.clip(jnp.minimum(b1x2, b2x2) - jnp.maximum(b1x1, b2x1), 0.0)
             * jnp.clip(jnp.minimum(b1y2, b2y2) - jnp.maximum(b1y1, b2y1), 0.0))
    union = pw * ph + tw * th - inter + EPS
    iou = inter / union
    cw = jnp.maximum(b1x2, b2x2) - jnp.minimum(b1x1, b2x1)
    ch = jnp.maximum(b1y2, b2y2) - jnp.minimum(b1y1, b2y1)
    c2 = cw * cw + ch * ch + EPS
    rho2 = (tx - px) ** 2 + (ty - py) ** 2
    v = (4.0 / np.pi ** 2) * (_atan_pos(tw / (th + EPS))
                              - _atan_pos(pw / (ph + EPS))) ** 2
    alpha = v / (v - iou + 1.0 + EPS)
    ciou = iou - (rho2 / c2 + v * alpha)
    abox[...] += (1.0 - ciou) * m

    @pl.when(step == n_steps - 1)
    def _final():
        n_pos = jnp.maximum(jnp.sum(anp[...]), 1.0)
        out_ref[0] = (jnp.sum(abox[...]) / n_pos
                      + jnp.sum(aobj[...]) / jnp.float32(n_obj)
                      + jnp.sum(acls[...]) / (n_pos * jnp.float32(n_cls)))


@jax.jit
def _yolo_loss(predictions, targets_in_grid, targets_masks, anchors):
    B, A, F, H, W = predictions.shape
    n_steps = B * A

    out = pl.pallas_call(
        functools.partial(_loss_kernel, n_steps=n_steps, n_obj=B * A * H * W,
                          n_cls=F - 5, n_anchors=A),
        grid=(n_steps,),
        in_specs=[
            pl.BlockSpec(memory_space=pl.ANY),
            pl.BlockSpec(memory_space=pl.ANY),
            pl.BlockSpec((1, 1, H, W), lambda i: (i // A, i % A, 0, 0)),
            pl.BlockSpec(memory_space=pltpu.SMEM),
        ],
        out_specs=pl.BlockSpec(memory_space=pltpu.SMEM),
        out_shape=jax.ShapeDtypeStruct((1,), jnp.float32),
        scratch_shapes=[pltpu.VMEM((H, W), jnp.float32),
                        pltpu.VMEM((H, W), jnp.float32),
                        pltpu.VMEM((H, W), jnp.float32),
                        pltpu.VMEM((H, W), jnp.float32),
                        pltpu.VMEM((2, F, H, W), jnp.float32),
                        pltpu.VMEM((2, F, H, W), jnp.float32),
                        pltpu.SemaphoreType.DMA((2,)),
                        pltpu.SemaphoreType.DMA((2,))],
    )(predictions, targets_in_grid, targets_masks, anchors)
    return out[0]


def kernel(predictions, targets_in_grid, targets_masks, anchors):
    return _yolo_loss(predictions, targets_in_grid, targets_masks, anchors)


# speedup vs baseline: 2.0474x; 2.0474x over previous
import functools

import jax
import jax.numpy as jnp
import numpy as np
from jax.experimental import pallas as pl
from jax.experimental.pallas import tpu as pltpu

ANCHOR_GAIN = 2.0
EPS = 1e-7


def _atan_pos(x):
    inv = x > 1.0
    r = jnp.where(inv, 1.0 / jnp.maximum(x, 1e-30), x)
    z = r * r
    p = 0.99997726 + z * (-0.33262347 + z * (0.19354346 + z * (
        -0.11643287 + z * (0.05265332 + z * -0.01172120))))
    a = r * p
    return jnp.where(inv, (np.pi / 2.0) - a, a)


def _loss_kernel(pred_ref, tgt_ref, mask_ref, anchors_ref, out_ref,
                 acc_ref, acc2_ref, *, n_blocks, n_obj, n_cls, blk_ba,
                 n_anchors, F):
    step = pl.program_id(0)

    @pl.when(step == 0)
    def _init():
        acc_ref[...] = jnp.zeros_like(acc_ref)
        acc2_ref[...] = jnp.zeros_like(acc2_ref)

    row = jax.lax.broadcasted_iota(jnp.int32, (8, F), 0)
    col = jax.lax.broadcasted_iota(jnp.int32, (8, F), 1)
    sel = jnp.where(row == 0, (col == 4).astype(jnp.float32),
                    jnp.where(row == 1, (col >= 5).astype(jnp.float32), 0.0))

    for j in range(blk_ba):
        z = pred_ref[j]
        t = tgt_ref[j]
        m = mask_ref[j].astype(jnp.float32)

        a = (step * blk_ba + j) % n_anchors
        aw = anchors_ref[a, 2]
        ah = anchors_ref[a, 3]

        zh = z.astype(jnp.bfloat16)
        th = t.astype(jnp.bfloat16)
        bce = (jnp.maximum(zh, jnp.bfloat16(0.0))
               + jnp.log1p(jnp.exp(-jnp.abs(zh)))) - zh * th
        two = jax.lax.dot_general(sel.astype(jnp.bfloat16), bce,
                                  (((1,), (0,)), ((), ())),
                                  preferred_element_type=jnp.float32)
        msel = jnp.where(row[:, :1] == 1, m, jnp.where(row[:, :1] == 0, 1.0, 0.0))
        acc_ref[...] += two * msel

        G = ANCHOR_GAIN
        px = jax.nn.sigmoid(z[0:1]) * G - (G - 1.0) / 2.0
        py = jax.nn.sigmoid(z[1:2]) * G - (G - 1.0) / 2.0
        pw = (jax.nn.sigmoid(z[2:3]) * G) ** 2 * aw
        ph = (jax.nn.sigmoid(z[3:4]) * G) ** 2 * ah
        tx = t[0:1]
        ty = t[1:2]
        tw = t[2:3]
        th = t[3:4]

        b1x1 = px - pw * 0.5
        b1x2 = px + pw * 0.5
        b1y1 = py - ph * 0.5
        b1y2 = py + ph * 0.5
        b2x1 = tx - tw * 0.5
        b2x2 = tx + tw * 0.5
        b2y1 = ty - th * 0.5
        b2y2 = ty + th * 0.5
        inter = (jnp.clip(jnp.minimum(b1x2, b2x2) - jnp.maximum(b1x1, b2x1), 0.0)
                 * jnp.clip(jnp.minimum(b1y2, b2y2) - jnp.maximum(b1y1, b2y1), 0.0))
        union = pw * ph + tw * th - inter + EPS
        iou = inter / union
        cw = jnp.maximum(b1x2, b2x2) - jnp.minimum(b1x1, b2x1)
        ch = jnp.maximum(b1y2, b2y2) - jnp.minimum(b1y1, b2y1)
        c2 = cw * cw + ch * ch + EPS
        rho2 = (tx - px) ** 2 + (ty - py) ** 2
        v = (4.0 / np.pi ** 2) * (_atan_pos(tw / (th + EPS))
                                  - _atan_pos(pw / (ph + EPS))) ** 2
        alpha = v / (v - iou + 1.0 + EPS)
        ciou = iou - (rho2 / c2 + v * alpha)
        acc2_ref[...] += jnp.concatenate([m, (1.0 - ciou) * m], axis=0)

    @pl.when(step == n_blocks - 1)
    def _final():
        n_pos = jnp.maximum(jnp.sum(acc2_ref[0:1]), 1.0)
        out_ref[0] = (jnp.sum(acc2_ref[1:2]) / n_pos
                      + jnp.sum(acc_ref[0:1]) / jnp.float32(n_obj)
                      + jnp.sum(acc_ref[1:2]) / (n_pos * jnp.float32(n_cls)))


@jax.jit
def _yolo_loss(predictions, targets_in_grid, targets_masks, anchors):
    B, A, F, H, W = predictions.shape
    BA, HW = B * A, H * W
    blk_ba = 2
    n_blocks = BA // blk_ba
    pred = predictions.reshape(BA, F, HW)
    tgt = targets_in_grid.reshape(BA, F, HW)
    mask = targets_masks.reshape(BA, 1, HW)

    out = pl.pallas_call(
        functools.partial(_loss_kernel, n_blocks=n_blocks, n_obj=BA * HW,
                          n_cls=F - 5, blk_ba=blk_ba, n_anchors=A, F=F),
        grid=(n_blocks,),
        in_specs=[
            pl.BlockSpec((blk_ba, F, HW), lambda i: (i, 0, 0)),
            pl.BlockSpec((blk_ba, F, HW), lambda i: (i, 0, 0)),
            pl.BlockSpec((blk_ba, 1, HW), lambda i: (i, 0, 0)),
            pl.BlockSpec(memory_space=pltpu.SMEM),
        ],
        out_specs=pl.BlockSpec(memory_space=pltpu.SMEM),
        out_shape=jax.ShapeDtypeStruct((1,), jnp.float32),
        scratch_shapes=[pltpu.VMEM((8, HW), jnp.float32),
                        pltpu.VMEM((2, HW), jnp.float32)],
    )(pred, tgt, mask, anchors)
    return out[0]


def kernel(predictions, targets_in_grid, targets_masks, anchors):
    return _yolo_loss(predictions, targets_in_grid, targets_masks, anchors)


# speedup vs baseline: 2.1607x; 1.0553x over previous
import functools

import jax
import jax.numpy as jnp
import numpy as np
from jax.experimental import pallas as pl
from jax.experimental.pallas import tpu as pltpu

ANCHOR_GAIN = 2.0
EPS = 1e-7


def _atan_pos(x):
    inv = x > 1.0
    r = jnp.where(inv, 1.0 / jnp.maximum(x, 1e-30), x)
    z = r * r
    p = 0.99997726 + z * (-0.33262347 + z * (0.19354346 + z * (
        -0.11643287 + z * (0.05265332 + z * -0.01172120))))
    a = r * p
    return jnp.where(inv, (np.pi / 2.0) - a, a)


def _loss_kernel(pred_ref, tgt_ref, mask_ref, anchors_ref, out_ref,
                 acc_ref, acc2_ref, *, n_blocks, n_obj, n_cls, blk_ba,
                 n_anchors, F):
    step = pl.program_id(0)

    @pl.when(step == 0)
    def _init():
        acc_ref[...] = jnp.zeros_like(acc_ref)
        acc2_ref[...] = jnp.zeros_like(acc2_ref)

    row = jax.lax.broadcasted_iota(jnp.int32, (8, F), 0)
    col = jax.lax.broadcasted_iota(jnp.int32, (8, F), 1)
    sel = jnp.where(row == 0, (col == 4).astype(jnp.float32),
                    jnp.where(row == 1, (col >= 5).astype(jnp.float32), 0.0))

    for j in range(blk_ba):
        z = pred_ref[j]
        t = tgt_ref[j]
        m = mask_ref[j].astype(jnp.float32)

        a = (step * blk_ba + j) % n_anchors
        aw = anchors_ref[a, 2]
        ah = anchors_ref[a, 3]

        zh = z.astype(jnp.bfloat16)
        th = t.astype(jnp.bfloat16)
        bce = (jnp.maximum(zh, jnp.bfloat16(0.0))
               + jnp.log1p(jnp.exp(-jnp.abs(zh)))) - zh * th
        two = jax.lax.dot_general(sel.astype(jnp.bfloat16), bce,
                                  (((1,), (0,)), ((), ())),
                                  preferred_element_type=jnp.float32)
        msel = jnp.where(row[:, :1] == 1, m, jnp.where(row[:, :1] == 0, 1.0, 0.0))
        acc_ref[...] += two * msel

        G = ANCHOR_GAIN
        px = jax.nn.sigmoid(z[0:1]) * G - (G - 1.0) / 2.0
        py = jax.nn.sigmoid(z[1:2]) * G - (G - 1.0) / 2.0
        pw = (jax.nn.sigmoid(z[2:3]) * G) ** 2 * aw
        ph = (jax.nn.sigmoid(z[3:4]) * G) ** 2 * ah
        tx = t[0:1]
        ty = t[1:2]
        tw = t[2:3]
        th = t[3:4]

        b1x1 = px - pw * 0.5
        b1x2 = px + pw * 0.5
        b1y1 = py - ph * 0.5
        b1y2 = py + ph * 0.5
        b2x1 = tx - tw * 0.5
        b2x2 = tx + tw * 0.5
        b2y1 = ty - th * 0.5
        b2y2 = ty + th * 0.5
        inter = (jnp.clip(jnp.minimum(b1x2, b2x2) - jnp.maximum(b1x1, b2x1), 0.0)
                 * jnp.clip(jnp.minimum(b1y2, b2y2) - jnp.maximum(b1y1, b2y1), 0.0))
        union = pw * ph + tw * th - inter + EPS
        iou = inter / union
        cw = jnp.maximum(b1x2, b2x2) - jnp.minimum(b1x1, b2x1)
        ch = jnp.maximum(b1y2, b2y2) - jnp.minimum(b1y1, b2y1)
        c2 = cw * cw + ch * ch + EPS
        rho2 = (tx - px) ** 2 + (ty - py) ** 2
        v = (4.0 / np.pi ** 2) * (_atan_pos(tw / (th + EPS))
                                  - _atan_pos(pw / (ph + EPS))) ** 2
        alpha = v / (v - iou + 1.0 + EPS)
        ciou = iou - (rho2 / c2 + v * alpha)
        acc2_ref[...] += jnp.concatenate([m, (1.0 - ciou) * m], axis=0)

    @pl.when(step == n_blocks - 1)
    def _final():
        n_pos = jnp.maximum(jnp.sum(acc2_ref[0:1]), 1.0)
        out_ref[0] = (jnp.sum(acc2_ref[1:2]) / n_pos
                      + jnp.sum(acc_ref[0:1]) / jnp.float32(n_obj)
                      + jnp.sum(acc_ref[1:2]) / (n_pos * jnp.float32(n_cls)))


@jax.jit
def _yolo_loss(predictions, targets_in_grid, targets_masks, anchors):
    B, A, F, H, W = predictions.shape
    BA, HW = B * A, H * W
    blk_ba = 6
    n_blocks = BA // blk_ba
    pred = predictions.reshape(BA, F, HW)
    tgt = targets_in_grid.reshape(BA, F, HW)
    mask = targets_masks.reshape(BA, 1, HW)

    out = pl.pallas_call(
        functools.partial(_loss_kernel, n_blocks=n_blocks, n_obj=BA * HW,
                          n_cls=F - 5, blk_ba=blk_ba, n_anchors=A, F=F),
        grid=(n_blocks,),
        in_specs=[
            pl.BlockSpec((blk_ba, F, HW), lambda i: (i, 0, 0)),
            pl.BlockSpec((blk_ba, F, HW), lambda i: (i, 0, 0)),
            pl.BlockSpec((blk_ba, 1, HW), lambda i: (i, 0, 0)),
            pl.BlockSpec(memory_space=pltpu.SMEM),
        ],
        out_specs=pl.BlockSpec(memory_space=pltpu.SMEM),
        out_shape=jax.ShapeDtypeStruct((1,), jnp.float32),
        scratch_shapes=[pltpu.VMEM((8, HW), jnp.float32),
                        pltpu.VMEM((2, HW), jnp.float32)],
    )(pred, tgt, mask, anchors)
    return out[0]


def kernel(predictions, targets_in_grid, targets_masks, anchors):
    return _yolo_loss(predictions, targets_in_grid, targets_masks, anchors)
